# baseline (speedup 1.0000x reference)
import functools

import jax
import jax.numpy as jnp
from jax import lax
from jax.experimental import pallas as pl
from jax.experimental.pallas import tpu as pltpu
from jax.experimental.pallas import tpu_sc as plsc

N = 10000
E = 320000
D = 128
H = 4
C = 32
OUT = H * C
NCLS = 16

NCORES = 2
NSUB = 16
NW = NCORES * NSUB
CH = 40
NBUF = 3
EPW = 10240
E_PAD = EPW * NW
NCHUNK = EPW // CH
HXW = 80
N_PAD = 10240
ROWS_PER_SUB = N_PAD // NSUB
DEN_W = 16
ACC_W = OUT + DEN_W

BN_ROWS = 2000
NBLK = N // BN_ROWS


def _pro_body(x_ref, wg_ref, bcat_ref, h_ref, asrc_ref, adst_ref):
    h = jnp.dot(x_ref[...], wg_ref[...], preferred_element_type=jnp.float32)
    h_ref[...] = h
    a = jnp.dot(h, bcat_ref[...], preferred_element_type=jnp.float32)
    asrc_ref[...] = a[:, :DEN_W]
    adst_ref[...] = a[:, DEN_W:]


def _prologue(x, W_gat, Bcat):
    return pl.pallas_call(
        _pro_body,
        grid=(NBLK,),
        in_specs=[
            pl.BlockSpec((BN_ROWS, D), lambda i: (i, 0)),
            pl.BlockSpec((D, D), lambda i: (0, 0)),
            pl.BlockSpec((D, 2 * DEN_W), lambda i: (0, 0)),
        ],
        out_specs=[
            pl.BlockSpec((BN_ROWS, D), lambda i: (i, 0)),
            pl.BlockSpec((BN_ROWS, DEN_W), lambda i: (i, 0)),
            pl.BlockSpec((BN_ROWS, DEN_W), lambda i: (i, 0)),
        ],
        out_shape=[
            jax.ShapeDtypeStruct((N, D), jnp.float32),
            jax.ShapeDtypeStruct((N, DEN_W), jnp.float32),
            jax.ShapeDtypeStruct((N_PAD, DEN_W), jnp.float32),
        ],
    )(x, W_gat, Bcat)


_sc_mesh = plsc.VectorSubcoreMesh(core_axis_name="c", subcore_axis_name="s")


_sc_scratch = []
for _b in range(NBUF):
    _sc_scratch += [
        pltpu.VMEM((CH,), jnp.int32),
        pltpu.VMEM((CH,), jnp.int32),
        pltpu.VMEM((CH, HXW), jnp.int32),
        pltpu.VMEM((CH, ACC_W), jnp.float32),
        pltpu.VMEM((CH, DEN_W), jnp.float32),
    ]
_sc_scratch += [
    pltpu.VMEM_SHARED((N_PAD, ACC_W), jnp.float32),
]
_sc_scratch += [pltpu.SemaphoreType.DMA] * (3 * NBUF)


@functools.partial(
    pl.kernel,
    out_type=jax.ShapeDtypeStruct((NCORES, N_PAD, ACC_W), jnp.float32),
    mesh=_sc_mesh,
    scratch_types=tuple(_sc_scratch),
    compiler_params=pltpu.CompilerParams(use_tc_tiling_on_sc=False),
)
def _sc_edge(src_hbm, dst_hbm, hx_hbm, adst_hbm, zacc_hbm,
             out_hbm, *scr):
    bufs = [scr[b * 5:(b + 1) * 5] for b in range(NBUF)]
    acc_sh = scr[5 * NBUF]
    semA = scr[5 * NBUF + 1:5 * NBUF + 1 + NBUF]
    semG = scr[5 * NBUF + 1 + NBUF:5 * NBUF + 1 + 2 * NBUF]
    semS = scr[5 * NBUF + 1 + 2 * NBUF:5 * NBUF + 1 + 3 * NBUF]

    cid = lax.axis_index("c")
    sid = lax.axis_index("s")
    wid = cid * NSUB + sid

    def idx_start(i, b):
        base = wid * EPW + i * CH
        sv, dv = bufs[b][0], bufs[b][1]
        pltpu.async_copy(src_hbm.at[pl.ds(base, CH)], sv, semA[b])
        pltpu.async_copy(dst_hbm.at[pl.ds(base, CH)], dv, semA[b])

    def idx_wait(b):
        sv, dv = bufs[b][0], bufs[b][1]
        pltpu.make_async_copy(src_hbm.at[pl.ds(0, CH)], sv, semA[b]).wait()
        pltpu.make_async_copy(dst_hbm.at[pl.ds(0, CH)], dv, semA[b]).wait()

    def g_start(b):
        sv, dv, rbf, _, adv = bufs[b]
        pltpu.async_copy(hx_hbm.at[sv], rbf, semG[b])
        pltpu.async_copy(adst_hbm.at[dv], adv, semG[b])

    def g_wait(b):
        sv, dv, rbf, _, adv = bufs[b]
        pltpu.make_async_copy(hx_hbm.at[sv], rbf, semG[b]).wait()
        pltpu.make_async_copy(adst_hbm.at[dv], adv, semG[b]).wait()

    def compute(b):
        _, _, rbf, rf32, adv = bufs[b]

        @plsc.parallel_loop(0, CH, 1, unroll=4)
        def _(e):
            asrc = rbf[e, pl.ds(OUT // 2, 16)].astype(jnp.float32) * (1.0 / 1048576.0)
            s = asrc + adv[e, :]
            w = jnp.exp(jnp.where(s >= 0.0, s, 0.2 * s))
            rf32[e, pl.ds(OUT, 16)] = w
            ws = w * (1.0 / 2048.0)
            for hh in range(H):
                wvec = jnp.full((16,), ws[hh])
                x = rbf[e, pl.ds(hh * 16, 16)]
                lo = lax.shift_right_arithmetic(lax.shift_left(x, 16), 16)
                hi = lax.shift_right_arithmetic(x, 16)
                rf32[e, pl.ds(hh * C, 16)] = lo.astype(jnp.float32) * wvec
                rf32[e, pl.ds(hh * C + 16, 16)] = hi.astype(jnp.float32) * wvec

    def s_start(b):
        _, dv, _, rf32, _ = bufs[b]
        pltpu.async_copy(rf32, acc_sh.at[dv], semS[b], add=True)

    def s_wait(b):
        _, dv, _, rf32, _ = bufs[b]
        pltpu.make_async_copy(rf32, acc_sh.at[dv], semS[b]).wait()

    idx_start(0, 0)
    idx_start(1, 1)
    rbase = sid * ROWS_PER_SUB
    pltpu.sync_copy(zacc_hbm.at[pl.ds(rbase, ROWS_PER_SUB)],
                    acc_sh.at[pl.ds(rbase, ROWS_PER_SUB)])
    plsc.subcore_barrier()

    idx_wait(0)
    g_start(0)
    idx_wait(1)
    g_start(1)
    g_wait(0)
    compute(0)
    s_start(0)
    idx_start(2, 2)

    def outer(p, _):
        for r in range(3):
            b = (1 + r) % 3
            bn = (2 + r) % 3
            bp = (0 + r) % 3
            j = 1 + p * 3 + r
            idx_wait(bn)
            g_start(bn)
            g_wait(b)
            compute(b)
            s_start(b)
            s_wait(bp)
            idx_start(j + 2, bp)
        return 0

    lax.fori_loop(0, (NCHUNK - 4) // 3, outer, 0)

    idx_wait(2)
    g_start(2)
    g_wait(1)
    compute(1)
    s_start(1)
    s_wait(0)
    idx_start(NCHUNK - 1, 0)
    idx_wait(0)
    g_start(0)
    g_wait(2)
    compute(2)
    s_start(2)
    s_wait(1)
    g_wait(0)
    compute(0)
    s_start(0)
    s_wait(2)
    s_wait(0)
    plsc.subcore_barrier()

    pltpu.sync_copy(acc_sh.at[pl.ds(rbase, ROWS_PER_SUB)],
                    out_hbm.at[cid, pl.ds(rbase, ROWS_PER_SUB)])


def _epi_body(x_ref, h_ref, asrc_ref, adst_ref, accp_ref, S_ref, P_ref,
              wres_ref, bgat_ref, bres_ref, bng_ref, bnb_ref, bnm_ref,
              bnv_ref, wcls_ref, bcls_ref, out_ref):
    s = asrc_ref[...][:, :H] + adst_ref[...][:, :H]
    w_self = jnp.exp(jnp.where(s >= 0.0, s, 0.2 * s))
    accs = accp_ref[0] + accp_ref[1]
    den = accs[:, OUT:OUT + H] + w_self
    wexp = jnp.dot(w_self, S_ref[...], preferred_element_type=jnp.float32)
    denexp = jnp.dot(den, S_ref[...], preferred_element_type=jnp.float32)
    acc = jnp.dot(accs[:, :OUT], P_ref[...], preferred_element_type=jnp.float32)
    out_un = acc + wexp * h_ref[...]
    gat = out_un / (denexp + 1e-16)
    y = gat + bgat_ref[...] + jnp.dot(
        x_ref[...], wres_ref[...], preferred_element_type=jnp.float32) + bres_ref[...]
    scale = bng_ref[...] * lax.rsqrt(bnv_ref[...] + 1e-5)
    y = (y - bnm_ref[...]) * scale + bnb_ref[...]
    y = jnp.maximum(y, 0.0)
    out_ref[...] = jnp.dot(
        y, wcls_ref[...], preferred_element_type=jnp.float32) + bcls_ref[...]


def _epilogue(x, h, asrcp, adstp, accp, S, P, W_res, b_gat, b_res,
              bn_g, bn_b, bn_m, bn_v, W_cls, b_cls):
    full2 = lambda shape: pl.BlockSpec(shape, lambda i: (0,) * len(shape))
    return pl.pallas_call(
        _epi_body,
        grid=(NBLK,),
        in_specs=[
            pl.BlockSpec((BN_ROWS, D), lambda i: (i, 0)),
            pl.BlockSpec((BN_ROWS, D), lambda i: (i, 0)),
            pl.BlockSpec((BN_ROWS, DEN_W), lambda i: (i, 0)),
            pl.BlockSpec((BN_ROWS, DEN_W), lambda i: (i, 0)),
            pl.BlockSpec((NCORES, BN_ROWS, ACC_W), lambda i: (0, i, 0)),
            full2((H, D)),
            full2((D, D)),
            full2((D, D)),
            full2((1, D)),
            full2((1, D)),
            full2((1, D)), full2((1, D)), full2((1, D)), full2((1, D)),
            full2((D, NCLS)),
            full2((1, NCLS)),
        ],
        out_specs=pl.BlockSpec((BN_ROWS, NCLS), lambda i: (i, 0)),
        out_shape=jax.ShapeDtypeStruct((N, NCLS), jnp.float32),
    )(x, h, asrcp, adstp, accp, S, P, W_res, b_gat.reshape(1, D),
      b_res.reshape(1, D), bn_g.reshape(1, D), bn_b.reshape(1, D),
      bn_m.reshape(1, D), bn_v.reshape(1, D), W_cls, b_cls.reshape(1, NCLS))


def kernel(x, edge_index, W_gat, att_src, att_dst, b_gat, W_res, b_res,
           bn_g, bn_b, bn_m, bn_v, W_cls, b_cls):
    f32 = jnp.float32
    eye = jnp.eye(H, dtype=f32)
    att_s = att_src.reshape(H, C)
    att_d = att_dst.reshape(H, C)
    B_s = (att_s[:, :, None] * eye[:, None, :]).reshape(D, H)
    B_d = (att_d[:, :, None] * eye[:, None, :]).reshape(D, H)
    pad = jnp.zeros((D, DEN_W - H), f32)
    Bcat = jnp.concatenate([B_s, pad, B_d, pad], axis=1)
    S = (jnp.ones((H, C, 1), f32) * eye[:, None, :]).reshape(D, H).T

    h, asrcp, adstp = _prologue(x, W_gat, Bcat)

    hq = jnp.clip(jnp.round(h * 2048.0), -32767.0, 32767.0).astype(jnp.int16)
    hb = jax.lax.bitcast_convert_type(
        hq.reshape(N, OUT // 2, 2), jnp.int32)
    ai = jnp.clip(jnp.round(asrcp[:, :H] * 1048576.0),
                  -1e9, 1e9).astype(jnp.int32)
    hx = jnp.concatenate(
        [hb, ai, jnp.zeros((N, HXW - OUT // 2 - H), jnp.int32)], axis=1)

    perm = []
    for blk in range(OUT // 32):
        perm += [32 * blk + 2 * i for i in range(16)]
        perm += [32 * blk + 2 * i + 1 for i in range(16)]
    P = jnp.zeros((D, D), f32).at[jnp.arange(D), jnp.array(perm)].set(1.0)

    npad = E_PAD - E
    src = jnp.concatenate([edge_index[0], jnp.zeros((npad,), jnp.int32)])
    dst = jnp.concatenate([edge_index[1],
                           jnp.full((npad,), N_PAD - 1, jnp.int32)])
    zacc = jnp.zeros((N_PAD, ACC_W), f32)
    accp = _sc_edge(src, dst, hx, adstp, zacc)

    return _epilogue(x, h, asrcp, adstp, accp, S, P, W_res, b_gat, b_res,
                     bn_g, bn_b, bn_m, bn_v, W_cls, b_cls)


# speedup vs baseline: 1.2493x; 1.2493x over previous
import functools

import jax
import jax.numpy as jnp
from jax import lax
from jax.experimental import pallas as pl
from jax.experimental.pallas import tpu as pltpu
from jax.experimental.pallas import tpu_sc as plsc

N = 10000
E = 320000
D = 128
H = 4
C = 32
OUT = H * C
NCLS = 16

NCORES = 2
NSUB = 16
NW = NCORES * NSUB
CH = 64
NBUF = 3
EPW = 10240
E_PAD = EPW * NW
NCHUNK = EPW // CH
HXW = 80
N_PAD = 10240
ROWS_PER_SUB = N_PAD // NSUB
DEN_W = 16

BN_ROWS = 2000
NBLK = N // BN_ROWS


def _pro_body(x_ref, wg_ref, bcat_ref, h_ref, asrc_ref, adst_ref):
    h = jnp.dot(x_ref[...], wg_ref[...], preferred_element_type=jnp.float32)
    h_ref[...] = h
    a = jnp.dot(h, bcat_ref[...], preferred_element_type=jnp.float32)
    asrc_ref[...] = a[:, :DEN_W]
    adst_ref[...] = a[:, DEN_W:]


def _prologue(x, W_gat, Bcat):
    return pl.pallas_call(
        _pro_body,
        grid=(NBLK,),
        in_specs=[
            pl.BlockSpec((BN_ROWS, D), lambda i: (i, 0)),
            pl.BlockSpec((D, D), lambda i: (0, 0)),
            pl.BlockSpec((D, 2 * DEN_W), lambda i: (0, 0)),
        ],
        out_specs=[
            pl.BlockSpec((BN_ROWS, D), lambda i: (i, 0)),
            pl.BlockSpec((BN_ROWS, DEN_W), lambda i: (i, 0)),
            pl.BlockSpec((BN_ROWS, DEN_W), lambda i: (i, 0)),
        ],
        out_shape=[
            jax.ShapeDtypeStruct((N, D), jnp.float32),
            jax.ShapeDtypeStruct((N, DEN_W), jnp.float32),
            jax.ShapeDtypeStruct((N_PAD, DEN_W), jnp.float32),
        ],
    )(x, W_gat, Bcat)


_sc_mesh = plsc.VectorSubcoreMesh(core_axis_name="c", subcore_axis_name="s")


_sc_scratch = []
for _b in range(NBUF):
    _sc_scratch += [
        pltpu.VMEM((CH,), jnp.int32),
        pltpu.VMEM((CH,), jnp.int32),
        pltpu.VMEM((CH, HXW), jnp.int32),
        pltpu.VMEM((CH, DEN_W), jnp.float32),
        pltpu.VMEM((CH, DEN_W), jnp.float32),
    ]
for _q in range(2):
    _sc_scratch += [
        pltpu.VMEM((CH, OUT), jnp.float32),
    ]
_sc_scratch += [
    pltpu.VMEM_SHARED((N_PAD, OUT), jnp.float32),
    pltpu.VMEM_SHARED((N_PAD, DEN_W), jnp.float32),
]
_sc_scratch += [pltpu.SemaphoreType.DMA] * (2 * NBUF + 2)


@functools.partial(
    pl.kernel,
    out_type=(
        jax.ShapeDtypeStruct((NCORES, N_PAD, OUT), jnp.float32),
        jax.ShapeDtypeStruct((NCORES, N_PAD, DEN_W), jnp.float32),
    ),
    mesh=_sc_mesh,
    scratch_types=tuple(_sc_scratch),
    compiler_params=pltpu.CompilerParams(use_tc_tiling_on_sc=False),
)
def _sc_edge(src_hbm, dst_hbm, hx_hbm, adst_hbm, zacc_hbm, zden_hbm,
             out_hbm, den_hbm, *scr):
    bufs = [scr[b * 5:(b + 1) * 5] for b in range(NBUF)]
    rf32s = scr[5 * NBUF:5 * NBUF + 2]
    acc_sh, den_sh = scr[5 * NBUF + 2], scr[5 * NBUF + 3]
    base_s = 5 * NBUF + 4
    semA = scr[base_s:base_s + NBUF]
    semG = scr[base_s + NBUF:base_s + 2 * NBUF]
    semS = scr[base_s + 2 * NBUF:base_s + 2 * NBUF + 2]

    cid = lax.axis_index("c")
    sid = lax.axis_index("s")
    wid = cid * NSUB + sid

    def idx_start(i, b):
        base = wid * EPW + i * CH
        sv, dv = bufs[b][0], bufs[b][1]
        pltpu.async_copy(src_hbm.at[pl.ds(base, CH)], sv, semA[b])
        pltpu.async_copy(dst_hbm.at[pl.ds(base, CH)], dv, semA[b])

    def idx_wait(b):
        sv, dv = bufs[b][0], bufs[b][1]
        pltpu.make_async_copy(src_hbm.at[pl.ds(0, CH)], sv, semA[b]).wait()
        pltpu.make_async_copy(dst_hbm.at[pl.ds(0, CH)], dv, semA[b]).wait()

    def g_start(b):
        sv, dv, rbf, adv, _ = bufs[b]
        pltpu.async_copy(hx_hbm.at[sv], rbf, semG[b])
        pltpu.async_copy(adst_hbm.at[dv], adv, semG[b])

    def g_wait(b):
        sv, dv, rbf, adv, _ = bufs[b]
        pltpu.make_async_copy(hx_hbm.at[sv], rbf, semG[b]).wait()
        pltpu.make_async_copy(adst_hbm.at[dv], adv, semG[b]).wait()

    def compute(b, q):
        _, _, rbf, adv, wv = bufs[b]
        rf32 = rf32s[q]

        @plsc.parallel_loop(0, CH, 1, unroll=4)
        def _(e):
            asrc = rbf[e, pl.ds(OUT // 2, 16)].astype(jnp.float32) * (1.0 / 1048576.0)
            s = asrc + adv[e, :]
            w = jnp.exp(jnp.where(s >= 0.0, s, 0.2 * s))
            wv[e, :] = w
            ws = w * (1.0 / 2048.0)
            for hh in range(H):
                wvec = jnp.full((16,), ws[hh])
                x = rbf[e, pl.ds(hh * 16, 16)]
                lo = lax.shift_right_arithmetic(lax.shift_left(x, 16), 16)
                hi = lax.shift_right_arithmetic(x, 16)
                rf32[e, pl.ds(hh * C, 16)] = lo.astype(jnp.float32) * wvec
                rf32[e, pl.ds(hh * C + 16, 16)] = hi.astype(jnp.float32) * wvec

    def s_start(b, q):
        _, dv, _, _, wv = bufs[b]
        pltpu.async_copy(rf32s[q], acc_sh.at[dv], semS[q], add=True)
        pltpu.async_copy(wv, den_sh.at[dv], semS[q], add=True)

    def s_wait(b, q):
        _, dv, _, _, wv = bufs[b]
        pltpu.make_async_copy(rf32s[q], acc_sh.at[dv], semS[q]).wait()
        pltpu.make_async_copy(wv, den_sh.at[dv], semS[q]).wait()

    idx_start(0, 0)
    idx_start(1, 1)
    rbase = sid * ROWS_PER_SUB
    pltpu.sync_copy(zacc_hbm.at[pl.ds(rbase, ROWS_PER_SUB)],
                    acc_sh.at[pl.ds(rbase, ROWS_PER_SUB)])
    pltpu.sync_copy(zden_hbm.at[pl.ds(rbase, ROWS_PER_SUB)],
                    den_sh.at[pl.ds(rbase, ROWS_PER_SUB)])
    plsc.subcore_barrier()

    idx_wait(0)
    g_start(0)
    idx_wait(1)
    g_start(1)
    g_wait(0)
    compute(0, 0)
    s_start(0, 0)
    idx_start(2, 2)

    def outer(p, _):
        for r in range(6):
            b = (1 + r) % 3
            bn = (2 + r) % 3
            bp = (0 + r) % 3
            q = (1 + r) % 2
            qp = (0 + r) % 2
            j = 1 + p * 6 + r
            idx_wait(bn)
            g_start(bn)
            g_wait(b)
            compute(b, q)
            s_start(b, q)
            s_wait(bp, qp)
            idx_start(j + 2, bp)
        return 0

    lax.fori_loop(0, (NCHUNK - 4) // 6, outer, 0)

    idx_wait(2)
    g_start(2)
    g_wait(1)
    compute(1, 1)
    s_start(1, 1)
    s_wait(0, 0)
    idx_start(NCHUNK - 1, 0)
    idx_wait(0)
    g_start(0)
    g_wait(2)
    compute(2, 0)
    s_start(2, 0)
    s_wait(1, 1)
    g_wait(0)
    compute(0, 1)
    s_start(0, 1)
    s_wait(2, 0)
    s_wait(0, 1)
    plsc.subcore_barrier()

    pltpu.sync_copy(acc_sh.at[pl.ds(rbase, ROWS_PER_SUB)],
                    out_hbm.at[cid, pl.ds(rbase, ROWS_PER_SUB)])
    pltpu.sync_copy(den_sh.at[pl.ds(rbase, ROWS_PER_SUB)],
                    den_hbm.at[cid, pl.ds(rbase, ROWS_PER_SUB)])


def _epi_body(x_ref, h_ref, asrc_ref, adst_ref, accp_ref, denp_ref, S_ref, P_ref,
              wres_ref, bgat_ref, bres_ref, bng_ref, bnb_ref, bnm_ref,
              bnv_ref, wcls_ref, bcls_ref, out_ref):
    s = asrc_ref[...][:, :H] + adst_ref[...][:, :H]
    w_self = jnp.exp(jnp.where(s >= 0.0, s, 0.2 * s))
    den = denp_ref[0][:, :H] + denp_ref[1][:, :H] + w_self
    wexp = jnp.dot(w_self, S_ref[...], preferred_element_type=jnp.float32)
    denexp = jnp.dot(den, S_ref[...], preferred_element_type=jnp.float32)
    acc_perm = accp_ref[0] + accp_ref[1]
    acc = jnp.dot(acc_perm, P_ref[...], preferred_element_type=jnp.float32)
    out_un = acc + wexp * h_ref[...]
    gat = out_un / (denexp + 1e-16)
    y = gat + bgat_ref[...] + jnp.dot(
        x_ref[...], wres_ref[...], preferred_element_type=jnp.float32) + bres_ref[...]
    scale = bng_ref[...] * lax.rsqrt(bnv_ref[...] + 1e-5)
    y = (y - bnm_ref[...]) * scale + bnb_ref[...]
    y = jnp.maximum(y, 0.0)
    out_ref[...] = jnp.dot(
        y, wcls_ref[...], preferred_element_type=jnp.float32) + bcls_ref[...]


def _epilogue(x, h, asrcp, adstp, accp, denp, S, P, W_res, b_gat, b_res,
              bn_g, bn_b, bn_m, bn_v, W_cls, b_cls):
    full2 = lambda shape: pl.BlockSpec(shape, lambda i: (0,) * len(shape))
    return pl.pallas_call(
        _epi_body,
        grid=(NBLK,),
        in_specs=[
            pl.BlockSpec((BN_ROWS, D), lambda i: (i, 0)),
            pl.BlockSpec((BN_ROWS, D), lambda i: (i, 0)),
            pl.BlockSpec((BN_ROWS, DEN_W), lambda i: (i, 0)),
            pl.BlockSpec((BN_ROWS, DEN_W), lambda i: (i, 0)),
            pl.BlockSpec((NCORES, BN_ROWS, OUT), lambda i: (0, i, 0)),
            pl.BlockSpec((NCORES, BN_ROWS, DEN_W), lambda i: (0, i, 0)),
            full2((H, D)),
            full2((D, D)),
            full2((D, D)),
            full2((1, D)),
            full2((1, D)),
            full2((1, D)), full2((1, D)), full2((1, D)), full2((1, D)),
            full2((D, NCLS)),
            full2((1, NCLS)),
        ],
        out_specs=pl.BlockSpec((BN_ROWS, NCLS), lambda i: (i, 0)),
        out_shape=jax.ShapeDtypeStruct((N, NCLS), jnp.float32),
    )(x, h, asrcp, adstp, accp, denp, S, P, W_res, b_gat.reshape(1, D),
      b_res.reshape(1, D), bn_g.reshape(1, D), bn_b.reshape(1, D),
      bn_m.reshape(1, D), bn_v.reshape(1, D), W_cls, b_cls.reshape(1, NCLS))


def kernel(x, edge_index, W_gat, att_src, att_dst, b_gat, W_res, b_res,
           bn_g, bn_b, bn_m, bn_v, W_cls, b_cls):
    f32 = jnp.float32
    eye = jnp.eye(H, dtype=f32)
    att_s = att_src.reshape(H, C)
    att_d = att_dst.reshape(H, C)
    B_s = (att_s[:, :, None] * eye[:, None, :]).reshape(D, H)
    B_d = (att_d[:, :, None] * eye[:, None, :]).reshape(D, H)
    pad = jnp.zeros((D, DEN_W - H), f32)
    Bcat = jnp.concatenate([B_s, pad, B_d, pad], axis=1)
    S = (jnp.ones((H, C, 1), f32) * eye[:, None, :]).reshape(D, H).T

    h, asrcp, adstp = _prologue(x, W_gat, Bcat)

    hq = jnp.clip(jnp.round(h * 2048.0), -32767.0, 32767.0).astype(jnp.int16)
    hb = jax.lax.bitcast_convert_type(
        hq.reshape(N, OUT // 2, 2), jnp.int32)
    ai = jnp.clip(jnp.round(asrcp[:, :H] * 1048576.0),
                  -1e9, 1e9).astype(jnp.int32)
    hx = jnp.concatenate(
        [hb, ai, jnp.zeros((N, HXW - OUT // 2 - H), jnp.int32)], axis=1)

    perm = []
    for blk in range(OUT // 32):
        perm += [32 * blk + 2 * i for i in range(16)]
        perm += [32 * blk + 2 * i + 1 for i in range(16)]
    P = jnp.zeros((D, D), f32).at[jnp.arange(D), jnp.array(perm)].set(1.0)

    npad = E_PAD - E
    src = jnp.concatenate([edge_index[0], jnp.zeros((npad,), jnp.int32)])
    dst = jnp.concatenate([edge_index[1],
                           jnp.full((npad,), N_PAD - 1, jnp.int32)])
    zacc = jnp.zeros((N_PAD, OUT), f32)
    zden = jnp.zeros((N_PAD, DEN_W), f32)
    accp, denp = _sc_edge(src, dst, hx, adstp, zacc, zden)

    return _epilogue(x, h, asrcp, adstp, accp, denp, S, P, W_res, b_gat, b_res,
                     bn_g, bn_b, bn_m, bn_v, W_cls, b_cls)
